# baseline (speedup 1.0000x reference)
import functools

import jax
import jax.numpy as jnp
from jax import lax
from jax.experimental import pallas as pl
from jax.experimental.pallas import tpu as pltpu
from jax.experimental.pallas import tpu_sc as plsc

N_ATOMS = 100000
D_FEAT = 128
N_SPECIES = 100

NUM_CORES = 2
NUM_SUBCORES = 16
NW = NUM_CORES * NUM_SUBCORES
LANES = 16

ROWS_PER_W = N_ATOMS // NW
WORDS_PER_W = ROWS_PER_W * D_FEAT
CHUNK_ROWS = 125
NUM_CHUNKS = ROWS_PER_W // CHUNK_ROWS
CHUNK_WORDS = CHUNK_ROWS * D_FEAT
VECS_PER_ROW = D_FEAT // LANES

Z_FETCH = ROWS_PER_W + 8
Z_FETCH = ((Z_FETCH + 7) // 8) * 8


def _sc_scale_rows(x_flat, z32, scales):
    mesh = plsc.VectorSubcoreMesh(
        core_axis_name="c",
        subcore_axis_name="s",
        num_cores=NUM_CORES,
        num_subcores=NUM_SUBCORES,
    )

    @functools.partial(
        pl.kernel,
        out_type=jax.ShapeDtypeStruct((N_ATOMS * D_FEAT,), jnp.float32),
        mesh=mesh,
        compiler_params=pltpu.CompilerParams(needs_layout_passes=False),
        scratch_types=[
            pltpu.VMEM((2, CHUNK_WORDS), jnp.float32),
            pltpu.VMEM((2, CHUNK_WORDS), jnp.float32),
            pltpu.VMEM((Z_FETCH,), jnp.int32),
            pltpu.VMEM((N_SPECIES,), jnp.float32),
            pltpu.VMEM((128,), jnp.float32),
            pltpu.SemaphoreType.DMA,
            pltpu.SemaphoreType.DMA,
            pltpu.SemaphoreType.DMA,
            pltpu.SemaphoreType.DMA,
        ],
    )
    def scale_kernel(x_hbm, z_hbm, scales_hbm, out_hbm, x_v, o_v, z_v, tab_v,
                     s_chunk, in_sem0, in_sem1, out_sem0, out_sem1):
        wid = lax.axis_index("s") * NUM_CORES + lax.axis_index("c")
        base = wid * WORDS_PER_W
        row0 = wid * ROWS_PER_W

        zbase = pl.multiple_of(row0 // 8 * 8, 8)
        zoff = row0 - zbase
        pltpu.sync_copy(scales_hbm, tab_v)
        pltpu.sync_copy(z_hbm.at[pl.ds(zbase, Z_FETCH)], z_v)

        in_sems = (in_sem0, in_sem1)
        out_sems = (out_sem0, out_sem1)

        def in_copy(c, slot):
            return pltpu.make_async_copy(
                x_hbm.at[pl.ds(base + c * CHUNK_WORDS, CHUNK_WORDS)],
                x_v.at[slot],
                in_sems[slot],
            )

        def out_copy(c, slot):
            return pltpu.make_async_copy(
                o_v.at[slot],
                out_hbm.at[pl.ds(base + c * CHUNK_WORDS, CHUNK_WORDS)],
                out_sems[slot],
            )

        def compute_chunk(c, slot):
            for i in range(8):
                idx = zoff + c * CHUNK_ROWS + i * LANES + lax.iota(jnp.int32, 16)
                z16 = plsc.load_gather(z_v, [idx])
                s16 = plsc.load_gather(tab_v, [z16])
                s_chunk[pl.ds(i * LANES, LANES)] = s16

            def row_body(r, carry):
                sidx = jnp.full((LANES,), r, dtype=jnp.int32)
                svec = plsc.load_gather(s_chunk, [sidx])
                for j in range(VECS_PER_ROW):
                    off = pl.multiple_of(r * D_FEAT + j * LANES, LANES)
                    o_v[slot, pl.ds(off, LANES)] = (
                        x_v[slot, pl.ds(off, LANES)] * svec
                    )
                return carry

            lax.fori_loop(0, CHUNK_ROWS, row_body, 0, unroll=5)

        in_copy(0, 0).start()
        for c in range(NUM_CHUNKS):
            slot = c % 2
            if c + 1 < NUM_CHUNKS:
                in_copy(c + 1, (c + 1) % 2).start()
            in_copy(c, slot).wait()
            if c >= 2:
                out_copy(c - 2, slot).wait()
            compute_chunk(c, slot)
            out_copy(c, slot).start()
        out_copy(NUM_CHUNKS - 2, (NUM_CHUNKS - 2) % 2).wait()
        out_copy(NUM_CHUNKS - 1, (NUM_CHUNKS - 1) % 2).wait()

    return scale_kernel(x_flat, z32, scales)


def kernel(x, Z, scales):
    z32 = Z.astype(jnp.int32)
    out_flat = _sc_scale_rows(x.reshape(-1), z32, scales)
    return out_flat.reshape(N_ATOMS, D_FEAT)


# speedup vs baseline: 1.6830x; 1.6830x over previous
import functools

import jax
import jax.numpy as jnp
from jax import lax
from jax.experimental import pallas as pl
from jax.experimental.pallas import tpu as pltpu
from jax.experimental.pallas import tpu_sc as plsc

N_ATOMS = 100000
D_FEAT = 128
N_SPECIES = 100

NUM_CORES = 2
NUM_SUBCORES = 16
NW = NUM_CORES * NUM_SUBCORES
LANES = 16

TAB_PAD = 128

B_PER_W = 4000
ACTIVE_W = N_ATOMS // B_PER_W

ROW_BLOCK = 10000


def _sc_gather_scales(z32, scales):
    mesh = plsc.VectorSubcoreMesh(
        core_axis_name="c",
        subcore_axis_name="s",
        num_cores=NUM_CORES,
        num_subcores=NUM_SUBCORES,
    )

    @functools.partial(
        pl.kernel,
        out_type=jax.ShapeDtypeStruct((N_ATOMS,), jnp.float32),
        mesh=mesh,
        compiler_params=pltpu.CompilerParams(needs_layout_passes=False),
        scratch_types=[
            pltpu.VMEM((B_PER_W,), jnp.int32),
            pltpu.VMEM((B_PER_W,), jnp.float32),
            pltpu.VMEM((N_SPECIES,), jnp.float32),
        ],
    )
    def gather_kernel(z_hbm, scales_hbm, out_hbm, idx_v, s_v, tab_v):
        wid = lax.axis_index("s") * NUM_CORES + lax.axis_index("c")

        @pl.when(wid < ACTIVE_W)
        def _():
            base = wid * B_PER_W
            pltpu.sync_copy(scales_hbm, tab_v)
            pltpu.sync_copy(z_hbm.at[pl.ds(base, B_PER_W)], idx_v)

            def body(i, carry):
                idx = idx_v[pl.ds(i * LANES, LANES)]
                s_v[pl.ds(i * LANES, LANES)] = plsc.load_gather(tab_v, [idx])
                return carry

            lax.fori_loop(0, B_PER_W // LANES, body, 0, unroll=4)
            pltpu.sync_copy(s_v, out_hbm.at[pl.ds(base, B_PER_W)])

    return gather_kernel(z32, scales)


def _tc_mul_kernel(x_ref, s_ref, out_ref):
    out_ref[...] = x_ref[...] * s_ref[...]


def _tc_scale(x, s2d):
    grid = (N_ATOMS // ROW_BLOCK,)
    return pl.pallas_call(
        _tc_mul_kernel,
        grid=grid,
        in_specs=[
            pl.BlockSpec((ROW_BLOCK, D_FEAT), lambda i: (i, 0)),
            pl.BlockSpec((ROW_BLOCK, 1), lambda i: (i, 0)),
        ],
        out_specs=pl.BlockSpec((ROW_BLOCK, D_FEAT), lambda i: (i, 0)),
        out_shape=jax.ShapeDtypeStruct((N_ATOMS, D_FEAT), jnp.float32),
    )(x, s2d)


def kernel(x, Z, scales):
    z32 = Z.astype(jnp.int32)
    s = _sc_gather_scales(z32, scales)
    return _tc_scale(x, s.reshape(N_ATOMS, 1))
